# Initial kernel scaffold; baseline (speedup 1.0000x reference)
import jax
import jax.numpy as jnp
from jax.experimental import pallas as pl


def kernel(X, vertex, edges, H, W):
    raise NotImplementedError("write your pallas kernel here")


# speedup vs baseline: 1.3435x; 1.3435x over previous
import functools

import jax
import jax.numpy as jnp
from jax import lax
from jax.experimental import pallas as pl
from jax.experimental.pallas import tpu as pltpu
from jax.experimental.pallas import tpu_sc as plsc

N = 10000
E = 4096
NNZ = 320000
D = 128

CHUNK = 128
TPW = 160
R = 16 * TPW

EPT = E // 16
CROWS = E // D
VH = 4096
NSUB = 3
NP = NSUB * VH
ACC = 4112

F32 = jnp.float32
I32 = jnp.int32


_B1 = 400
_G1 = N // _B1


def _k1_body(x_ref, w_ref, h_ref, xw_ref, degv_ref, dege_ref):
    i = pl.program_id(0)
    xw_ref[...] = jnp.dot(x_ref[...], w_ref[...], preferred_element_type=F32)

    h = h_ref[...]
    rv = jnp.sum(h, axis=1, keepdims=True)
    r = lax.rsqrt(rv)
    degv_ref[...] = jnp.where(jnp.isinf(r), 1.0, r)

    ce = jnp.sum(h, axis=0, keepdims=True)

    @pl.when(i == 0)
    def _():
        dege_ref[...] = jnp.zeros_like(dege_ref)

    dege_ref[...] += ce

    @pl.when(i == _G1 - 1)
    def _():
        dege_ref[...] = lax.rsqrt(dege_ref[...])


def _kernel1(X, W, H):
    return pl.pallas_call(
        _k1_body,
        grid=(_G1,),
        in_specs=[
            pl.BlockSpec((_B1, D), lambda i: (i, 0)),
            pl.BlockSpec((D, D), lambda i: (0, 0)),
            pl.BlockSpec((_B1, E), lambda i: (i, 0)),
        ],
        out_specs=[
            pl.BlockSpec((_B1, D), lambda i: (i, 0)),
            pl.BlockSpec((_B1, 1), lambda i: (i, 0)),
            pl.BlockSpec((1, E), lambda i: (0, 0)),
        ],
        out_shape=[
            jax.ShapeDtypeStruct((N, D), F32),
            jax.ShapeDtypeStruct((N, 1), F32),
            jax.ShapeDtypeStruct((1, E), F32),
        ],
    )(X, W, H)


_MESH = dict(core_axis_name="c", subcore_axis_name="s", num_cores=1)
_ZR = 264


def _bcast(vec16, k):
    return vec16.at[jnp.full((16,), k, I32)].get(mode="promise_in_bounds")


def _sc_mega(pidx, table, dege, degv):
    mesh = plsc.VectorSubcoreMesh(**_MESH)

    @functools.partial(
        pl.kernel,
        out_type=(
            jax.ShapeDtypeStruct((NP, D), F32),
            jax.ShapeDtypeStruct((E, D), F32),
        ),
        mesh=mesh,
        compiler_params=pltpu.CompilerParams(needs_layout_passes=False),
        scratch_types=[
            pltpu.VMEM((TPW, CHUNK), I32),
            pltpu.VMEM((TPW, CHUNK), I32),
            pltpu.VMEM((CHUNK, D), F32),
            pltpu.VMEM((_ZR, D), F32),
            pltpu.VMEM((CROWS, D), F32),
            pltpu.VMEM((EPT,), F32),
            pltpu.VMEM((EPT,), F32),
            pltpu.VMEM_SHARED((ACC, D), F32),
            pltpu.SemaphoreType.DMA,
            pltpu.SemaphoreType.DMA,
        ],
    )
    def mega(pidx_hbm, table_hbm, dege_hbm, degv_hbm,
             xv_hbm, xep_hbm,
             gidx, sidx, gbuf, stage, hist, dege_v, degv_v,
             acc, gsem, ssem):
        s = lax.axis_index("s")
        base = s * TPW
        pltpu.sync_copy(pidx_hbm.at[pl.ds(base, TPW)], gidx)

        def unpack(t, carry):
            for j in range(CHUNK // 16):
                pk = gidx[t, pl.ds(j * 16, 16)]
                sidx[t, pl.ds(j * 16, 16)] = pk & 8191
                gidx[t, pl.ds(j * 16, 16)] = jnp.minimum(pk >> 13, N - 1)
            return carry

        lax.fori_loop(0, TPW, unpack, 0)

        def zstage(i, carry):
            for j in range(D // 16):
                stage[i, pl.ds(j * 16, 16)] = jnp.zeros((16,), F32)
            return carry

        def zero_acc():
            @pl.when(s < 15)
            def _():
                pltpu.sync_copy(stage, acc.at[pl.ds(s * 264, 264)])

            @pl.when(s == 15)
            def _():
                pltpu.sync_copy(stage.at[pl.ds(0, 152)],
                                acc.at[pl.ds(3960, 152)])

        lax.fori_loop(0, _ZR, zstage, 0)
        zero_acc()

        def zhist(i, carry):
            for j in range(D // 16):
                hist[i, pl.ds(j * 16, 16)] = jnp.zeros((16,), F32)
            return carry

        lax.fori_loop(0, CROWS, zhist, 0)
        plsc.subcore_barrier()

        def body_a(t, carry):
            pltpu.async_copy(table_hbm.at[gidx.at[t]], gbuf, gsem).wait()
            pltpu.async_copy(gbuf, acc.at[sidx.at[t]], ssem, add=True).wait()
            for j in range(CHUNK // 16):
                idx16 = sidx[t, pl.ds(j * 16, 16)]
                plsc.addupdate_scatter(
                    hist, [idx16 >> 7, idx16 & 127], jnp.ones((16,), F32),
                    mask=idx16 < E)
            return carry

        lax.fori_loop(0, TPW, body_a, 0)
        plsc.subcore_barrier()

        pltpu.sync_copy(hist, xv_hbm.at[pl.ds(s * CROWS, CROWS)])
        e0 = s * EPT
        pltpu.sync_copy(acc.at[pl.ds(e0, EPT)], stage.at[pl.ds(0, EPT)])
        pltpu.sync_copy(dege_hbm.at[pl.ds(e0, EPT)], dege_v)
        plsc.subcore_barrier()

        @pl.when(s < 4)
        def _():
            for r in range(8):
                for j in range(D // 16):
                    hist[r, pl.ds(j * 16, 16)] = jnp.zeros((16,), F32)
            for src in range(16):
                pltpu.sync_copy(xv_hbm.at[pl.ds(src * CROWS + 8 * s, 8)],
                                hist.at[pl.ds(8, 8)])
                for r in range(8):
                    for j in range(D // 16):
                        hist[r, pl.ds(j * 16, 16)] = (
                            hist[r, pl.ds(j * 16, 16)]
                            + hist[8 + r, pl.ds(j * 16, 16)])
            pltpu.sync_copy(hist.at[pl.ds(0, 8)],
                            xv_hbm.at[pl.ds(16 * CROWS + 8 * s, 8)])

        plsc.subcore_barrier()
        pltpu.sync_copy(xv_hbm.at[pl.ds(16 * CROWS, CROWS)], hist)

        def scale_stage_rows(nblocks, scale_fn):
            def blk(b, carry):
                scale16 = scale_fn(b)
                for k in range(16):
                    vec = _bcast(scale16, k)
                    r = b * 16 + k
                    for j in range(D // 16):
                        stage[r, pl.ds(j * 16, 16)] = (
                            stage[r, pl.ds(j * 16, 16)] * vec)
                return carry

            lax.fori_loop(0, nblocks, blk, 0)

        def ascale(b):
            e = e0 + b * 16
            cnt16 = hist[e >> 7, pl.ds(e & 127, 16)]
            return dege_v[pl.ds(b * 16, 16)] / jnp.maximum(cnt16, 1.0)

        scale_stage_rows(EPT // 16, ascale)
        pltpu.sync_copy(stage.at[pl.ds(0, EPT)], xep_hbm.at[pl.ds(e0, EPT)])

        def clampe(t, carry):
            for j in range(CHUNK // 16):
                e16 = sidx[t, pl.ds(j * 16, 16)]
                sidx[t, pl.ds(j * 16, 16)] = jnp.minimum(e16, E - 1)
            return carry

        lax.fori_loop(0, TPW, clampe, 0)

        def remap(p):
            pltpu.sync_copy(pidx_hbm.at[pl.ds(base, TPW)], gidx)

            def rm(t, carry):
                for j in range(CHUNK // 16):
                    v16 = gidx[t, pl.ds(j * 16, 16)] >> 13
                    n16 = v16 - p * VH
                    ok = jnp.logical_and(n16 >= 0, n16 < VH)
                    gidx[t, pl.ds(j * 16, 16)] = jnp.where(ok, n16, VH)
                return carry

            lax.fori_loop(0, TPW, rm, 0)

        def body_b(t, carry):
            pltpu.async_copy(xep_hbm.at[sidx.at[t]], gbuf, gsem).wait()
            pltpu.async_copy(gbuf, acc.at[gidx.at[t]], ssem, add=True).wait()
            return carry

        def copy_out(p):
            r0 = s * EPT
            pltpu.sync_copy(acc.at[pl.ds(r0, EPT)], stage.at[pl.ds(0, EPT)])
            pltpu.sync_copy(degv_hbm.at[pl.ds(p * VH + r0, EPT)], degv_v)
            scale_stage_rows(EPT // 16, lambda b: degv_v[pl.ds(b * 16, 16)])
            pltpu.sync_copy(stage.at[pl.ds(0, EPT)],
                            xv_hbm.at[pl.ds(p * VH + r0, EPT)])

        for p in range(NSUB):
            remap(p)
            lax.fori_loop(0, _ZR, zstage, 0)
            zero_acc()
            plsc.subcore_barrier()
            lax.fori_loop(0, TPW, body_b, 0)
            plsc.subcore_barrier()
            copy_out(p)
            plsc.subcore_barrier()

    return mega(pidx, table, dege, degv)


def kernel(X, vertex, edges, H, W):
    pad = R * CHUNK - NNZ
    vertex = vertex.astype(I32)
    edges = edges.astype(I32)
    packed = vertex * 8192 + edges
    packed = jnp.reshape(
        jnp.concatenate([packed, jnp.full((pad,), N * 8192 + E, I32)]),
        (R, CHUNK))

    xw, degv, dege_row = _kernel1(X, W, H)
    dege = jnp.reshape(dege_row, (E,))
    degv_pad = jnp.pad(jnp.reshape(degv, (N,)), (0, NP - N))

    xv, _ = _sc_mega(packed, xw, dege, degv_pad)
    return xv[:N]


# speedup vs baseline: 1.4773x; 1.0996x over previous
import functools

import jax
import jax.numpy as jnp
from jax import lax
from jax.experimental import pallas as pl
from jax.experimental.pallas import tpu as pltpu
from jax.experimental.pallas import tpu_sc as plsc

N = 10000
E = 4096
NNZ = 320000
D = 128

CHUNK = 128
TPW = 160
R = 16 * TPW

EPT = E // 16
CROWS = E // D
VH = 4096
NSUB = 3
NP = NSUB * VH
ACC = 4112

F32 = jnp.float32
I32 = jnp.int32


_B1 = 400
_G1 = N // _B1


def _k1_body(x_ref, w_ref, h_ref, xw_ref, degv_ref, dege_ref):
    i = pl.program_id(0)
    xw_ref[...] = jnp.dot(x_ref[...], w_ref[...], preferred_element_type=F32)

    h = h_ref[...]
    rv = jnp.sum(h, axis=1, keepdims=True)
    r = lax.rsqrt(rv)
    degv_ref[...] = jnp.where(jnp.isinf(r), 1.0, r)

    ce = jnp.sum(h, axis=0, keepdims=True)

    @pl.when(i == 0)
    def _():
        dege_ref[...] = jnp.zeros_like(dege_ref)

    dege_ref[...] += ce

    @pl.when(i == _G1 - 1)
    def _():
        dege_ref[...] = lax.rsqrt(dege_ref[...])


def _kernel1(X, W, H):
    return pl.pallas_call(
        _k1_body,
        grid=(_G1,),
        in_specs=[
            pl.BlockSpec((_B1, D), lambda i: (i, 0)),
            pl.BlockSpec((D, D), lambda i: (0, 0)),
            pl.BlockSpec((_B1, E), lambda i: (i, 0)),
        ],
        out_specs=[
            pl.BlockSpec((_B1, D), lambda i: (i, 0)),
            pl.BlockSpec((_B1, 1), lambda i: (i, 0)),
            pl.BlockSpec((1, E), lambda i: (0, 0)),
        ],
        out_shape=[
            jax.ShapeDtypeStruct((N, D), F32),
            jax.ShapeDtypeStruct((N, 1), F32),
            jax.ShapeDtypeStruct((1, E), F32),
        ],
    )(X, W, H)


_MESH = dict(core_axis_name="c", subcore_axis_name="s", num_cores=1)


def _bcast(vec16, k):
    return vec16.at[jnp.full((16,), k, I32)].get(mode="promise_in_bounds")


def _sc_mega(pidx, table, dege, degv):
    mesh = plsc.VectorSubcoreMesh(**_MESH)

    @functools.partial(
        pl.kernel,
        out_type=(
            jax.ShapeDtypeStruct((NP, D), F32),
            jax.ShapeDtypeStruct((E, D), F32),
        ),
        mesh=mesh,
        compiler_params=pltpu.CompilerParams(needs_layout_passes=False),
        scratch_types=[
            pltpu.VMEM((TPW, CHUNK), I32),
            pltpu.VMEM((8, CHUNK), I32),
            pltpu.VMEM((8, CHUNK), I32),
            pltpu.VMEM((2, CHUNK, D), F32),
            pltpu.VMEM((EPT, D), F32),
            pltpu.VMEM((CROWS, D), F32),
            pltpu.VMEM((EPT,), F32),
            pltpu.VMEM((EPT,), F32),
            pltpu.VMEM_SHARED((ACC, D), F32),
            pltpu.SemaphoreType.DMA,
            pltpu.SemaphoreType.DMA,
        ],
    )
    def mega(pidx_hbm, table_hbm, dege_hbm, degv_hbm,
             xv_hbm, xep_hbm,
             pix, idxg, idxs, gbuf, stage, hist, dege_v, degv_v,
             acc, gsem, ssem):
        s = lax.axis_index("s")
        base = s * TPW
        pltpu.sync_copy(pidx_hbm.at[pl.ds(base, TPW)], pix)

        def zstage(i, carry):
            for j in range(D // 16):
                stage[i, pl.ds(j * 16, 16)] = jnp.zeros((16,), F32)
            return carry

        def zero_acc():
            @pl.when(s < 15)
            def _():
                pltpu.sync_copy(stage, acc.at[pl.ds(s * 264, 256)])
                pltpu.sync_copy(stage.at[pl.ds(0, 8)],
                                acc.at[pl.ds(s * 264 + 256, 8)])

            @pl.when(s == 15)
            def _():
                pltpu.sync_copy(stage.at[pl.ds(0, 152)],
                                acc.at[pl.ds(3960, 152)])

        lax.fori_loop(0, EPT, zstage, 0)
        zero_acc()

        def zhist(i, carry):
            for j in range(D // 16):
                hist[i, pl.ds(j * 16, 16)] = jnp.zeros((16,), F32)
            return carry

        lax.fori_loop(0, CROWS, zhist, 0)
        plsc.subcore_barrier()

        def wait_g(t):
            pltpu.make_async_copy(
                table_hbm.at[idxg.at[t & 7]], gbuf.at[t & 1], gsem).wait()

        def fire_s(t):
            pltpu.async_copy(gbuf.at[t & 1], acc.at[idxs.at[t & 7]], ssem,
                             add=True)

        def wait_s(t):
            pltpu.make_async_copy(
                gbuf.at[t & 1], acc.at[idxs.at[t & 7]], ssem).wait()

        def prep_a(t):
            w = t & 7
            for j in range(CHUNK // 16):
                pk = pix[t, pl.ds(j * 16, 16)]
                idxs[w, pl.ds(j * 16, 16)] = pk & 8191
                idxg[w, pl.ds(j * 16, 16)] = jnp.minimum(pk >> 13, N - 1)

        def fire_g_a(t):
            pltpu.async_copy(table_hbm.at[idxg.at[t & 7]], gbuf.at[t & 1],
                             gsem)

        prep_a(0)
        fire_g_a(0)

        def body_a(t, carry):
            wait_g(t)
            fire_s(t)

            @pl.when(t >= 1)
            def _():
                wait_s(t - 1)

            @pl.when(t + 1 < TPW)
            def _():
                prep_a(t + 1)
                fire_g_a(t + 1)

            for j in range(CHUNK // 16):
                idx16 = pix[t, pl.ds(j * 16, 16)] & 8191
                plsc.addupdate_scatter(
                    hist, [idx16 >> 7, idx16 & 127], jnp.ones((16,), F32),
                    mask=idx16 < E)
            return carry

        lax.fori_loop(0, TPW, body_a, 0)
        wait_s(TPW - 1)
        plsc.subcore_barrier()

        pltpu.sync_copy(hist, xv_hbm.at[pl.ds(s * CROWS, CROWS)])
        e0 = s * EPT
        pltpu.sync_copy(acc.at[pl.ds(e0, EPT)], stage)
        pltpu.sync_copy(dege_hbm.at[pl.ds(e0, EPT)], dege_v)
        plsc.subcore_barrier()

        @pl.when(s < 4)
        def _():
            for r in range(8):
                for j in range(D // 16):
                    hist[r, pl.ds(j * 16, 16)] = jnp.zeros((16,), F32)
            for src in range(16):
                pltpu.sync_copy(xv_hbm.at[pl.ds(src * CROWS + 8 * s, 8)],
                                hist.at[pl.ds(8, 8)])
                for r in range(8):
                    for j in range(D // 16):
                        hist[r, pl.ds(j * 16, 16)] = (
                            hist[r, pl.ds(j * 16, 16)]
                            + hist[8 + r, pl.ds(j * 16, 16)])
            pltpu.sync_copy(hist.at[pl.ds(0, 8)],
                            xv_hbm.at[pl.ds(16 * CROWS + 8 * s, 8)])

        plsc.subcore_barrier()
        pltpu.sync_copy(xv_hbm.at[pl.ds(16 * CROWS, CROWS)], hist)

        def scale_stage_rows(scale_fn):
            def blk(b, carry):
                scale16 = scale_fn(b)
                for k in range(16):
                    vec = _bcast(scale16, k)
                    r = b * 16 + k
                    for j in range(D // 16):
                        stage[r, pl.ds(j * 16, 16)] = (
                            stage[r, pl.ds(j * 16, 16)] * vec)
                return carry

            lax.fori_loop(0, EPT // 16, blk, 0)

        def ascale(b):
            e = e0 + b * 16
            cnt16 = hist[e >> 7, pl.ds(e & 127, 16)]
            return dege_v[pl.ds(b * 16, 16)] / jnp.maximum(cnt16, 1.0)

        scale_stage_rows(ascale)
        pltpu.sync_copy(stage, xep_hbm.at[pl.ds(e0, EPT)])

        def copy_out(p):
            r0 = s * EPT
            pltpu.sync_copy(acc.at[pl.ds(r0, EPT)], stage)
            pltpu.sync_copy(degv_hbm.at[pl.ds(p * VH + r0, EPT)], degv_v)
            scale_stage_rows(lambda b: degv_v[pl.ds(b * 16, 16)])
            pltpu.sync_copy(stage, xv_hbm.at[pl.ds(p * VH + r0, EPT)])

        for p in range(NSUB):
            def prep_b(t, p=p):
                w = t & 7
                for j in range(CHUNK // 16):
                    pk = pix[t, pl.ds(j * 16, 16)]
                    e16 = pk & 8191
                    n16 = (pk >> 13) - p * VH
                    ok = jnp.logical_and(n16 >= 0, n16 < VH)
                    idxg[w, pl.ds(j * 16, 16)] = jnp.minimum(e16, E - 1)
                    idxs[w, pl.ds(j * 16, 16)] = jnp.where(ok, n16, VH)

            def fire_g_b(t):
                pltpu.async_copy(xep_hbm.at[idxg.at[t & 7]], gbuf.at[t & 1],
                                 gsem)

            def wait_g_b(t):
                pltpu.make_async_copy(
                    xep_hbm.at[idxg.at[t & 7]], gbuf.at[t & 1], gsem).wait()

            lax.fori_loop(0, EPT, zstage, 0)
            zero_acc()
            plsc.subcore_barrier()

            prep_b(0)
            fire_g_b(0)

            def body_b(t, carry, fire_g_b=fire_g_b, wait_g_b=wait_g_b,
                       prep_b=prep_b):
                wait_g_b(t)
                fire_s(t)

                @pl.when(t >= 1)
                def _():
                    wait_s(t - 1)

                @pl.when(t + 1 < TPW)
                def _():
                    prep_b(t + 1)
                    fire_g_b(t + 1)

                return carry

            lax.fori_loop(0, TPW, body_b, 0)
            wait_s(TPW - 1)
            plsc.subcore_barrier()
            copy_out(p)
            plsc.subcore_barrier()

    return mega(pidx, table, dege, degv)


def kernel(X, vertex, edges, H, W):
    pad = R * CHUNK - NNZ
    vertex = vertex.astype(I32)
    edges = edges.astype(I32)
    packed = vertex * 8192 + edges
    packed = jnp.reshape(
        jnp.concatenate([packed, jnp.full((pad,), N * 8192 + E, I32)]),
        (R, CHUNK))

    xw, degv, dege_row = _kernel1(X, W, H)
    dege = jnp.reshape(dege_row, (E,))
    degv_pad = jnp.pad(jnp.reshape(degv, (N,)), (0, NP - N))

    xv, _ = _sc_mega(packed, xw, dege, degv_pad)
    return xv[:N]


# speedup vs baseline: 1.8960x; 1.2834x over previous
import functools

import jax
import jax.numpy as jnp
from jax import lax
from jax.experimental import pallas as pl
from jax.experimental.pallas import tpu as pltpu
from jax.experimental.pallas import tpu_sc as plsc

N = 10000
E = 4096
NNZ = 320000
D = 128

CHUNK = 128
TPW = 80
R = 32 * TPW

EPT = E // 16
CROWS = E // D
VH = 4096
NSUB = 3
NP = NSUB * VH
ACC = 4112

F32 = jnp.float32
I32 = jnp.int32

_MESH = dict(core_axis_name="c", subcore_axis_name="s")


_B1 = 400
_G1 = N // _B1


def _k1_body(x_ref, w_ref, h_ref, xw_ref, degv_ref, dege_ref):
    i = pl.program_id(0)
    xw_ref[...] = jnp.dot(x_ref[...], w_ref[...], preferred_element_type=F32)

    h = h_ref[...]
    rv = jnp.sum(h, axis=1, keepdims=True)
    r = lax.rsqrt(rv)
    degv_ref[...] = jnp.where(jnp.isinf(r), 1.0, r)

    ce = jnp.sum(h, axis=0, keepdims=True)

    @pl.when(i == 0)
    def _():
        dege_ref[...] = jnp.zeros_like(dege_ref)

    dege_ref[...] += ce

    @pl.when(i == _G1 - 1)
    def _():
        dege_ref[...] = lax.rsqrt(dege_ref[...])


def _kernel1(X, W, H):
    return pl.pallas_call(
        _k1_body,
        grid=(_G1,),
        in_specs=[
            pl.BlockSpec((_B1, D), lambda i: (i, 0)),
            pl.BlockSpec((D, D), lambda i: (0, 0)),
            pl.BlockSpec((_B1, E), lambda i: (i, 0)),
        ],
        out_specs=[
            pl.BlockSpec((_B1, D), lambda i: (i, 0)),
            pl.BlockSpec((_B1, 1), lambda i: (i, 0)),
            pl.BlockSpec((1, E), lambda i: (0, 0)),
        ],
        out_shape=[
            jax.ShapeDtypeStruct((N, D), F32),
            jax.ShapeDtypeStruct((N, 1), F32),
            jax.ShapeDtypeStruct((1, E), F32),
        ],
    )(X, W, H)


def _sc_phase_a(pidx, table):
    mesh = plsc.VectorSubcoreMesh(**_MESH)

    @functools.partial(
        pl.kernel,
        out_type=(
            jax.ShapeDtypeStruct((2, E, D), F32),
            jax.ShapeDtypeStruct((2, CROWS, D), F32),
        ),
        mesh=mesh,
        compiler_params=pltpu.CompilerParams(needs_layout_passes=False),
        scratch_types=[
            pltpu.VMEM((TPW, CHUNK), I32),
            pltpu.VMEM((8, CHUNK), I32),
            pltpu.VMEM((8, CHUNK), I32),
            pltpu.VMEM((2, CHUNK, D), F32),
            pltpu.VMEM((EPT, D), F32),
            pltpu.VMEM((CROWS, D), F32),
            pltpu.VMEM_SHARED((ACC, D), F32),
            pltpu.SemaphoreType.DMA,
            pltpu.SemaphoreType.DMA,
        ],
    )
    def ka(pidx_hbm, table_hbm, sums_hbm, cnt_hbm,
           pix, idxg, idxs, gbuf, stage, hist, acc, gsem, ssem):
        c = lax.axis_index("c")
        s = lax.axis_index("s")
        base = (c * 16 + s) * TPW
        pltpu.sync_copy(pidx_hbm.at[pl.ds(base, TPW)], pix)

        def zstage(i, carry):
            for j in range(D // 16):
                stage[i, pl.ds(j * 16, 16)] = jnp.zeros((16,), F32)
            return carry

        lax.fori_loop(0, EPT, zstage, 0)

        @pl.when(s < 15)
        def _():
            pltpu.sync_copy(stage, acc.at[pl.ds(s * 264, 256)])
            pltpu.sync_copy(stage.at[pl.ds(0, 8)],
                            acc.at[pl.ds(s * 264 + 256, 8)])

        @pl.when(s == 15)
        def _():
            pltpu.sync_copy(stage.at[pl.ds(0, 152)], acc.at[pl.ds(3960, 152)])

        def zhist(i, carry):
            for j in range(D // 16):
                hist[i, pl.ds(j * 16, 16)] = jnp.zeros((16,), F32)
            return carry

        lax.fori_loop(0, CROWS, zhist, 0)
        plsc.subcore_barrier()

        def prep(t):
            w = t & 7
            for j in range(CHUNK // 16):
                pk = pix[t, pl.ds(j * 16, 16)]
                idxs[w, pl.ds(j * 16, 16)] = pk & 8191
                idxg[w, pl.ds(j * 16, 16)] = jnp.minimum(pk >> 13, N - 1)

        def fire_g(t):
            pltpu.async_copy(table_hbm.at[idxg.at[t & 7]], gbuf.at[t & 1],
                             gsem)

        def wait_g(t):
            pltpu.make_async_copy(
                table_hbm.at[idxg.at[t & 7]], gbuf.at[t & 1], gsem).wait()

        def fire_s(t):
            pltpu.async_copy(gbuf.at[t & 1], acc.at[idxs.at[t & 7]], ssem,
                             add=True)

        def wait_s(t):
            pltpu.make_async_copy(
                gbuf.at[t & 1], acc.at[idxs.at[t & 7]], ssem).wait()

        prep(0)
        fire_g(0)

        def body(t, carry):
            wait_g(t)
            fire_s(t)

            @pl.when(t >= 1)
            def _():
                wait_s(t - 1)

            @pl.when(t + 1 < TPW)
            def _():
                prep(t + 1)
                fire_g(t + 1)

            for j in range(CHUNK // 16):
                idx16 = pix[t, pl.ds(j * 16, 16)] & 8191
                plsc.addupdate_scatter(
                    hist, [idx16 >> 7, idx16 & 127], jnp.ones((16,), F32),
                    mask=idx16 < E)
            return carry

        lax.fori_loop(0, TPW, body, 0)
        wait_s(TPW - 1)
        plsc.subcore_barrier()

        pltpu.sync_copy(hist, sums_hbm.at[c, pl.ds(s * CROWS, CROWS)])
        plsc.subcore_barrier()

        @pl.when(s < 4)
        def _():
            for r in range(8):
                for j in range(D // 16):
                    hist[r, pl.ds(j * 16, 16)] = jnp.zeros((16,), F32)
            for src in range(16):
                pltpu.sync_copy(
                    sums_hbm.at[c, pl.ds(src * CROWS + 8 * s, 8)],
                    hist.at[pl.ds(8, 8)])
                for r in range(8):
                    for j in range(D // 16):
                        hist[r, pl.ds(j * 16, 16)] = (
                            hist[r, pl.ds(j * 16, 16)]
                            + hist[8 + r, pl.ds(j * 16, 16)])
            pltpu.sync_copy(hist.at[pl.ds(0, 8)],
                            cnt_hbm.at[c, pl.ds(8 * s, 8)])

        plsc.subcore_barrier()

        pltpu.sync_copy(acc.at[pl.ds(s * EPT, EPT)], stage)
        pltpu.sync_copy(stage, sums_hbm.at[c, pl.ds(s * EPT, EPT)])

    return ka(pidx, table)


def _sc_phase_b(pidx, xep):
    mesh = plsc.VectorSubcoreMesh(**_MESH)

    @functools.partial(
        pl.kernel,
        out_type=jax.ShapeDtypeStruct((2, NP, D), F32),
        mesh=mesh,
        compiler_params=pltpu.CompilerParams(needs_layout_passes=False),
        scratch_types=[
            pltpu.VMEM((TPW, CHUNK), I32),
            pltpu.VMEM((8, CHUNK), I32),
            pltpu.VMEM((8, CHUNK), I32),
            pltpu.VMEM((2, CHUNK, D), F32),
            pltpu.VMEM((EPT, D), F32),
            pltpu.VMEM_SHARED((ACC, D), F32),
            pltpu.SemaphoreType.DMA,
            pltpu.SemaphoreType.DMA,
        ],
    )
    def kb(pidx_hbm, xep_hbm, out_hbm,
           pix, idxg, idxs, gbuf, stage, acc, gsem, ssem):
        c = lax.axis_index("c")
        s = lax.axis_index("s")
        base = (c * 16 + s) * TPW
        pltpu.sync_copy(pidx_hbm.at[pl.ds(base, TPW)], pix)

        def zstage(i, carry):
            for j in range(D // 16):
                stage[i, pl.ds(j * 16, 16)] = jnp.zeros((16,), F32)
            return carry

        def fire_s(t):
            pltpu.async_copy(gbuf.at[t & 1], acc.at[idxs.at[t & 7]], ssem,
                             add=True)

        def wait_s(t):
            pltpu.make_async_copy(
                gbuf.at[t & 1], acc.at[idxs.at[t & 7]], ssem).wait()

        def fire_g(t):
            pltpu.async_copy(xep_hbm.at[idxg.at[t & 7]], gbuf.at[t & 1],
                             gsem)

        def wait_g(t):
            pltpu.make_async_copy(
                xep_hbm.at[idxg.at[t & 7]], gbuf.at[t & 1], gsem).wait()

        for p in range(NSUB):
            def prep(t, p=p):
                w = t & 7
                for j in range(CHUNK // 16):
                    pk = pix[t, pl.ds(j * 16, 16)]
                    e16 = pk & 8191
                    n16 = (pk >> 13) - p * VH
                    ok = jnp.logical_and(n16 >= 0, n16 < VH)
                    idxg[w, pl.ds(j * 16, 16)] = jnp.minimum(e16, E - 1)
                    idxs[w, pl.ds(j * 16, 16)] = jnp.where(ok, n16, VH)

            lax.fori_loop(0, EPT, zstage, 0)

            @pl.when(s < 15)
            def _():
                pltpu.sync_copy(stage, acc.at[pl.ds(s * 264, 256)])
                pltpu.sync_copy(stage.at[pl.ds(0, 8)],
                                acc.at[pl.ds(s * 264 + 256, 8)])

            @pl.when(s == 15)
            def _():
                pltpu.sync_copy(stage.at[pl.ds(0, 152)],
                                acc.at[pl.ds(3960, 152)])

            plsc.subcore_barrier()

            prep(0)
            fire_g(0)

            def body(t, carry, prep=prep):
                wait_g(t)
                fire_s(t)

                @pl.when(t >= 1)
                def _():
                    wait_s(t - 1)

                @pl.when(t + 1 < TPW)
                def _():
                    prep(t + 1)
                    fire_g(t + 1)

                return carry

            lax.fori_loop(0, TPW, body, 0)
            wait_s(TPW - 1)
            plsc.subcore_barrier()

            pltpu.sync_copy(acc.at[pl.ds(s * EPT, EPT)], stage)
            pltpu.sync_copy(stage,
                            out_hbm.at[c, pl.ds(p * VH + s * EPT, EPT)])
            plsc.subcore_barrier()

    return kb(pidx, xep)


def _ca_body(p_ref, c0_ref, c1_ref, dege_ref, xep_ref):
    sums = p_ref[0] + p_ref[1]
    cnt = c0_ref[...] + c1_ref[...]
    xep_ref[...] = sums * (dege_ref[...] / jnp.maximum(cnt, 1.0))


def _combine_a(pa, c0, c1, dege):
    return pl.pallas_call(
        _ca_body,
        out_shape=jax.ShapeDtypeStruct((E, D), F32),
    )(pa, c0, c1, dege)


def _cb_body(pb_ref, degv_ref, out_ref):
    out_ref[...] = (pb_ref[0, 0:N, :] + pb_ref[1, 0:N, :]) * degv_ref[...]


def _combine_b(pb, degv):
    return pl.pallas_call(
        _cb_body,
        out_shape=jax.ShapeDtypeStruct((N, D), F32),
    )(pb, degv)


def kernel(X, vertex, edges, H, W):
    pad = R * CHUNK - NNZ
    vertex = vertex.astype(I32)
    edges = edges.astype(I32)
    packed = vertex * 8192 + edges
    packed = jnp.reshape(
        jnp.concatenate([packed, jnp.full((pad,), N * 8192 + E, I32)]),
        (R, CHUNK))

    xw, degv, dege_row = _kernel1(X, W, H)
    dege = jnp.reshape(dege_row, (E, 1))

    pa, cnt = _sc_phase_a(packed, xw)
    cnt = jnp.reshape(cnt, (2, E, 1))
    xep = _combine_a(pa, cnt[0], cnt[1], dege)
    pb = _sc_phase_b(packed, xep)
    return _combine_b(pb, degv)


# speedup vs baseline: 2.4815x; 1.3088x over previous
import functools

import jax
import jax.numpy as jnp
from jax import lax
from jax.experimental import pallas as pl
from jax.experimental.pallas import tpu as pltpu
from jax.experimental.pallas import tpu_sc as plsc

N = 10000
E = 4096
NNZ = 320000
D = 128

CHUNK = 128
TPW = 80
R = 32 * TPW

EPT = E // 16
CROWS = E // D
VH = 5120
NSUB = 2
NP = NSUB * VH
ACC = 4112
ACCB = 5248
VPT = VH // 16

F32 = jnp.float32
I32 = jnp.int32

_MESH = dict(core_axis_name="c", subcore_axis_name="s")


_B1 = 400
_G1 = N // _B1


def _k1_body(x_ref, w_ref, h_ref, xw_ref, degv_ref, dege_ref):
    i = pl.program_id(0)
    xw_ref[...] = jnp.dot(x_ref[...], w_ref[...], preferred_element_type=F32)

    h = h_ref[...]
    rv = jnp.sum(h, axis=1, keepdims=True)
    r = lax.rsqrt(rv)
    degv_ref[...] = jnp.where(jnp.isinf(r), 1.0, r)

    ce = jnp.sum(h, axis=0, keepdims=True)

    @pl.when(i == 0)
    def _():
        dege_ref[...] = jnp.zeros_like(dege_ref)

    dege_ref[...] += ce

    @pl.when(i == _G1 - 1)
    def _():
        dege_ref[...] = lax.rsqrt(dege_ref[...])


def _kernel1(X, W, H):
    return pl.pallas_call(
        _k1_body,
        grid=(_G1,),
        in_specs=[
            pl.BlockSpec((_B1, D), lambda i: (i, 0)),
            pl.BlockSpec((D, D), lambda i: (0, 0)),
            pl.BlockSpec((_B1, E), lambda i: (i, 0)),
        ],
        out_specs=[
            pl.BlockSpec((_B1, D), lambda i: (i, 0)),
            pl.BlockSpec((_B1, 1), lambda i: (i, 0)),
            pl.BlockSpec((1, E), lambda i: (0, 0)),
        ],
        out_shape=[
            jax.ShapeDtypeStruct((N, D), F32),
            jax.ShapeDtypeStruct((N, 1), F32),
            jax.ShapeDtypeStruct((1, E), F32),
        ],
    )(X, W, H)


def _sc_phase_a(pidx, table):
    mesh = plsc.VectorSubcoreMesh(**_MESH)

    @functools.partial(
        pl.kernel,
        out_type=(
            jax.ShapeDtypeStruct((2, E, D), F32),
            jax.ShapeDtypeStruct((2, CROWS, D), F32),
        ),
        mesh=mesh,
        compiler_params=pltpu.CompilerParams(needs_layout_passes=False),
        scratch_types=[
            pltpu.VMEM((TPW, CHUNK), I32),
            pltpu.VMEM((8, CHUNK), I32),
            pltpu.VMEM((8, CHUNK), I32),
            pltpu.VMEM((2, CHUNK, D), F32),
            pltpu.VMEM((EPT, D), F32),
            pltpu.VMEM((CROWS, D), F32),
            pltpu.VMEM_SHARED((ACC, D), F32),
            pltpu.SemaphoreType.DMA,
            pltpu.SemaphoreType.DMA,
        ],
    )
    def ka(pidx_hbm, table_hbm, sums_hbm, cnt_hbm,
           pix, idxg, idxs, gbuf, stage, hist, acc, gsem, ssem):
        c = lax.axis_index("c")
        s = lax.axis_index("s")
        base = (c * 16 + s) * TPW
        pltpu.sync_copy(pidx_hbm.at[pl.ds(base, TPW)], pix)

        def zstage(i, carry):
            for j in range(D // 16):
                stage[i, pl.ds(j * 16, 16)] = jnp.zeros((16,), F32)
            return carry

        lax.fori_loop(0, EPT, zstage, 0)

        @pl.when(s < 15)
        def _():
            pltpu.sync_copy(stage, acc.at[pl.ds(s * 264, 256)])
            pltpu.sync_copy(stage.at[pl.ds(0, 8)],
                            acc.at[pl.ds(s * 264 + 256, 8)])

        @pl.when(s == 15)
        def _():
            pltpu.sync_copy(stage.at[pl.ds(0, 152)], acc.at[pl.ds(3960, 152)])

        def zhist(i, carry):
            for j in range(D // 16):
                hist[i, pl.ds(j * 16, 16)] = jnp.zeros((16,), F32)
            return carry

        lax.fori_loop(0, CROWS, zhist, 0)
        plsc.subcore_barrier()

        def prep(t):
            w = t & 7
            for j in range(CHUNK // 16):
                pk = pix[t, pl.ds(j * 16, 16)]
                idxs[w, pl.ds(j * 16, 16)] = pk & 8191
                idxg[w, pl.ds(j * 16, 16)] = jnp.minimum(pk >> 13, N - 1)

        def fire_g(t):
            pltpu.async_copy(table_hbm.at[idxg.at[t & 7]], gbuf.at[t & 1],
                             gsem)

        def wait_g(t):
            pltpu.make_async_copy(
                table_hbm.at[idxg.at[t & 7]], gbuf.at[t & 1], gsem).wait()

        def fire_s(t):
            pltpu.async_copy(gbuf.at[t & 1], acc.at[idxs.at[t & 7]], ssem,
                             add=True)

        def wait_s(t):
            pltpu.make_async_copy(
                gbuf.at[t & 1], acc.at[idxs.at[t & 7]], ssem).wait()

        prep(0)
        fire_g(0)

        def body(t, carry):
            wait_g(t)
            fire_s(t)

            @pl.when(t >= 1)
            def _():
                wait_s(t - 1)

            @pl.when(t + 1 < TPW)
            def _():
                prep(t + 1)
                fire_g(t + 1)

            for j in range(CHUNK // 16):
                idx16 = pix[t, pl.ds(j * 16, 16)] & 8191
                plsc.addupdate_scatter(
                    hist, [idx16 >> 7, idx16 & 127], jnp.ones((16,), F32),
                    mask=idx16 < E)
            return carry

        lax.fori_loop(0, TPW, body, 0)
        wait_s(TPW - 1)
        plsc.subcore_barrier()

        pltpu.sync_copy(hist, sums_hbm.at[c, pl.ds(s * CROWS, CROWS)])
        plsc.subcore_barrier()

        @pl.when(s < 4)
        def _():
            for r in range(8):
                for j in range(D // 16):
                    hist[r, pl.ds(j * 16, 16)] = jnp.zeros((16,), F32)
            for src in range(16):
                pltpu.sync_copy(
                    sums_hbm.at[c, pl.ds(src * CROWS + 8 * s, 8)],
                    hist.at[pl.ds(8, 8)])
                for r in range(8):
                    for j in range(D // 16):
                        hist[r, pl.ds(j * 16, 16)] = (
                            hist[r, pl.ds(j * 16, 16)]
                            + hist[8 + r, pl.ds(j * 16, 16)])
            pltpu.sync_copy(hist.at[pl.ds(0, 8)],
                            cnt_hbm.at[c, pl.ds(8 * s, 8)])

        plsc.subcore_barrier()

        pltpu.sync_copy(acc.at[pl.ds(s * EPT, EPT)], stage)
        pltpu.sync_copy(stage, sums_hbm.at[c, pl.ds(s * EPT, EPT)])

    return ka(pidx, table)


def _sc_phase_b(pidx, xep):
    mesh = plsc.VectorSubcoreMesh(**_MESH)

    @functools.partial(
        pl.kernel,
        out_type=jax.ShapeDtypeStruct((2, NP, D), F32),
        mesh=mesh,
        compiler_params=pltpu.CompilerParams(needs_layout_passes=False),
        scratch_types=[
            pltpu.VMEM((TPW, CHUNK), I32),
            pltpu.VMEM((8, CHUNK), I32),
            pltpu.VMEM((8, CHUNK), I32),
            pltpu.VMEM((2, CHUNK, D), F32),
            pltpu.VMEM((VPT, D), F32),
            pltpu.VMEM_SHARED((ACCB, D), F32),
            pltpu.SemaphoreType.DMA,
            pltpu.SemaphoreType.DMA,
        ],
    )
    def kb(pidx_hbm, xep_hbm, out_hbm,
           pix, idxg, idxs, gbuf, stage, acc, gsem, ssem):
        c = lax.axis_index("c")
        s = lax.axis_index("s")
        base = (c * 16 + s) * TPW
        pltpu.sync_copy(pidx_hbm.at[pl.ds(base, TPW)], pix)

        def zstage(i, carry):
            for j in range(D // 16):
                stage[i, pl.ds(j * 16, 16)] = jnp.zeros((16,), F32)
            return carry

        def fire_s(t):
            pltpu.async_copy(gbuf.at[t & 1], acc.at[idxs.at[t & 7]], ssem,
                             add=True)

        def wait_s(t):
            pltpu.make_async_copy(
                gbuf.at[t & 1], acc.at[idxs.at[t & 7]], ssem).wait()

        def fire_g(t):
            pltpu.async_copy(xep_hbm.at[idxg.at[t & 7]], gbuf.at[t & 1],
                             gsem)

        def wait_g(t):
            pltpu.make_async_copy(
                xep_hbm.at[idxg.at[t & 7]], gbuf.at[t & 1], gsem).wait()

        for p in range(NSUB):
            def prep(t, p=p):
                w = t & 7
                for j in range(CHUNK // 16):
                    pk = pix[t, pl.ds(j * 16, 16)]
                    e16 = pk & 8191
                    n16 = (pk >> 13) - p * VH
                    ok = jnp.logical_and(n16 >= 0, n16 < VH)
                    idxg[w, pl.ds(j * 16, 16)] = jnp.minimum(e16, E - 1)
                    idxs[w, pl.ds(j * 16, 16)] = jnp.where(ok, n16, VH)

            lax.fori_loop(0, VPT, zstage, 0)
            pltpu.sync_copy(stage, acc.at[pl.ds(s * 328, 320)])
            pltpu.sync_copy(stage.at[pl.ds(0, 8)],
                            acc.at[pl.ds(s * 328 + 320, 8)])
            plsc.subcore_barrier()

            prep(0)
            fire_g(0)

            def body(t, carry, prep=prep):
                wait_g(t)
                fire_s(t)

                @pl.when(t >= 1)
                def _():
                    wait_s(t - 1)

                @pl.when(t + 1 < TPW)
                def _():
                    prep(t + 1)
                    fire_g(t + 1)

                return carry

            lax.fori_loop(0, TPW, body, 0)
            wait_s(TPW - 1)
            plsc.subcore_barrier()

            pltpu.sync_copy(acc.at[pl.ds(s * VPT, VPT)], stage)
            pltpu.sync_copy(stage,
                            out_hbm.at[c, pl.ds(p * VH + s * VPT, VPT)])
            plsc.subcore_barrier()

    return kb(pidx, xep)


def _ca_body(p_ref, c0_ref, c1_ref, dege_ref, xep_ref):
    sums = p_ref[0] + p_ref[1]
    cnt = c0_ref[...] + c1_ref[...]
    xep_ref[...] = sums * (dege_ref[...] / jnp.maximum(cnt, 1.0))


def _combine_a(pa, c0, c1, dege):
    return pl.pallas_call(
        _ca_body,
        out_shape=jax.ShapeDtypeStruct((E, D), F32),
    )(pa, c0, c1, dege)


def _cb_body(pb_ref, degv_ref, out_ref):
    out_ref[...] = (pb_ref[0, 0:N, :] + pb_ref[1, 0:N, :]) * degv_ref[...]


def _combine_b(pb, degv):
    return pl.pallas_call(
        _cb_body,
        out_shape=jax.ShapeDtypeStruct((N, D), F32),
    )(pb, degv)


def kernel(X, vertex, edges, H, W):
    pad = R * CHUNK - NNZ
    vertex = vertex.astype(I32)
    edges = edges.astype(I32)
    packed = vertex * 8192 + edges
    packed = jnp.reshape(
        jnp.concatenate([packed, jnp.full((pad,), N * 8192 + E, I32)]),
        (R, CHUNK))

    xw, degv, dege_row = _kernel1(X, W, H)
    dege = jnp.reshape(dege_row, (E, 1))

    pa, cnt = _sc_phase_a(packed, xw)
    cnt = jnp.reshape(cnt, (2, E, 1))
    xep = _combine_a(pa, cnt[0], cnt[1], dege)
    pb = _sc_phase_b(packed, xep)
    return _combine_b(pb, degv)
